# baseline (speedup 1.0000x reference)
import functools
import jax
import jax.numpy as jnp
from jax import lax
from jax.experimental import pallas as pl
from jax.experimental.pallas import tpu as pltpu

N = 10000
E = 160000
D = 256
NPAD = 10240
ROW_BLK = 256
EDGE_BLK = 1280
F32 = jnp.float32


def _dot_t(a, w):
    return lax.dot_general(a, w, (((1,), (1,)), ((), ())),
                           preferred_element_type=F32)


def _mm_body(x_ref, w_ref, b_ref, o_ref):
    o_ref[...] = _dot_t(x_ref[...], w_ref[...]) + b_ref[...]


def _mm(x, w, b):
    rows, k = x.shape
    out_d = w.shape[0]
    return pl.pallas_call(
        _mm_body,
        grid=(rows // ROW_BLK,),
        in_specs=[
            pl.BlockSpec((ROW_BLK, k), lambda i: (i, 0)),
            pl.BlockSpec((out_d, k), lambda i: (0, 0)),
            pl.BlockSpec((1, out_d), lambda i: (0, 0)),
        ],
        out_specs=pl.BlockSpec((ROW_BLK, out_d), lambda i: (i, 0)),
        out_shape=jax.ShapeDtypeStruct((rows, out_d), F32),
    )(x, w, b)


def _node_body(z_ref, g_ref, bln_ref, wc1_ref, bc1_ref, w2_ref, b2_ref,
               u_ref, v_ref, bout_ref):
    z = z_ref[...]
    mu = jnp.mean(z, axis=1, keepdims=True)
    zc = z - mu
    var = jnp.mean(zc * zc, axis=1, keepdims=True)
    h = jnp.maximum(zc * lax.rsqrt(var + 1e-5) * g_ref[...] + bln_ref[...], 0.0)
    cat = _dot_t(h, wc1_ref[...]) + bc1_ref[...]
    u_ref[...] = cat[:, :D]
    v_ref[...] = cat[:, D:2 * D]
    t = jnp.maximum(cat[:, 2 * D:], 0.0)
    bout_ref[...] = z + _dot_t(t, w2_ref[...]) + b2_ref[...]


def _node_stage(z, ln_g, ln_b, wc1, bc1, w2, b2):
    rows = z.shape[0]
    return pl.pallas_call(
        _node_body,
        grid=(rows // ROW_BLK,),
        in_specs=[
            pl.BlockSpec((ROW_BLK, D), lambda i: (i, 0)),
            pl.BlockSpec((1, D), lambda i: (0, 0)),
            pl.BlockSpec((1, D), lambda i: (0, 0)),
            pl.BlockSpec((3 * D, D), lambda i: (0, 0)),
            pl.BlockSpec((1, 3 * D), lambda i: (0, 0)),
            pl.BlockSpec((D, D), lambda i: (0, 0)),
            pl.BlockSpec((1, D), lambda i: (0, 0)),
        ],
        out_specs=[
            pl.BlockSpec((ROW_BLK, D), lambda i: (i, 0)),
            pl.BlockSpec((ROW_BLK, D), lambda i: (i, 0)),
            pl.BlockSpec((ROW_BLK, D), lambda i: (i, 0)),
        ],
        out_shape=[
            jax.ShapeDtypeStruct((rows, D), F32),
            jax.ShapeDtypeStruct((rows, D), F32),
            jax.ShapeDtypeStruct((rows, D), F32),
        ],
    )(z, ln_g, ln_b, wc1, bc1, w2, b2)


def _edge_body(g_ref, ea_ref, w1e_ref, b1e_ref, w2e_ref, b2e_ref,
               w2m_ref, b2m_ref, o_ref):
    gh = jnp.maximum(_dot_t(ea_ref[...], w1e_ref[...]) + b1e_ref[...], 0.0)
    gate = _dot_t(gh, w2e_ref[...]) + b2e_ref[...]
    msgpre = _dot_t(g_ref[...], w2m_ref[...]) + b2m_ref[...]
    o_ref[...] = gate * msgpre


def _edge_stage(g, ea_pad, w1e, b1e, w2e, b2e, w2m, b2m):
    ek = ea_pad.shape[1]
    return pl.pallas_call(
        _edge_body,
        grid=(E // EDGE_BLK,),
        in_specs=[
            pl.BlockSpec((EDGE_BLK, D), lambda i: (i, 0)),
            pl.BlockSpec((EDGE_BLK, ek), lambda i: (i, 0)),
            pl.BlockSpec((D, ek), lambda i: (0, 0)),
            pl.BlockSpec((1, D), lambda i: (0, 0)),
            pl.BlockSpec((D, D), lambda i: (0, 0)),
            pl.BlockSpec((1, D), lambda i: (0, 0)),
            pl.BlockSpec((D, D), lambda i: (0, 0)),
            pl.BlockSpec((1, D), lambda i: (0, 0)),
        ],
        out_specs=pl.BlockSpec((EDGE_BLK, D), lambda i: (i, 0)),
        out_shape=jax.ShapeDtypeStruct((E, D), F32),
    )(g, ea_pad, w1e, b1e, w2e, b2e, w2m, b2m)


def _final_body(z_ref, w_ref, b_ref, wc_ref, bc_ref, o_ref):
    gc = _dot_t(z_ref[...], w_ref[...]) + b_ref[...]
    cat = _dot_t(gc, wc_ref[...]) + bc_ref[...]
    u = jax.nn.sigmoid(cat[:, :D])
    hc = jnp.maximum(cat[:, D:], 0.0)
    o_ref[...] = (1.0 - u) * hc


def _final_stage(z, w, b, wc, bc):
    rows = z.shape[0]
    return pl.pallas_call(
        _final_body,
        grid=(rows // ROW_BLK,),
        in_specs=[
            pl.BlockSpec((ROW_BLK, D), lambda i: (i, 0)),
            pl.BlockSpec((D, D), lambda i: (0, 0)),
            pl.BlockSpec((1, D), lambda i: (0, 0)),
            pl.BlockSpec((2 * D, D), lambda i: (0, 0)),
            pl.BlockSpec((1, 2 * D), lambda i: (0, 0)),
        ],
        out_specs=pl.BlockSpec((ROW_BLK, D), lambda i: (i, 0)),
        out_shape=jax.ShapeDtypeStruct((rows, D), F32),
    )(z, w, b, wc, bc)


def kernel(x, edge_index, edge_attr, params):
    src = edge_index[0]
    dst = edge_index[1]

    x_pad = jnp.pad(x, ((0, NPAD - N), (0, 0)))
    ea_pad = jnp.pad(edge_attr, ((0, 0), (0, 128 - edge_attr.shape[1])))

    cnt = jax.ops.segment_sum(jnp.ones((E,), F32), dst, num_segments=N)
    has_in = (cnt > 0)[:, None]

    def row(v):
        return v.reshape(1, -1)

    z0 = _mm(x_pad, params['lin1_W'], row(params['lin1_b']))

    def conv_layer(z, p):
        c = p['conv']
        m1 = c['node_mlp1']
        m2 = c['node_mlp2']
        wc1 = jnp.concatenate([m1['W1'][:, :D], m1['W1'][:, D:], m2['W1']], 0)
        bc1 = jnp.concatenate([m1['b1'], jnp.zeros((D,), F32), m2['b1']])
        u_a, v_a, b_a = _node_stage(z, row(p['ln_g']), row(p['ln_b']),
                                    wc1, row(bc1), m2['W2'], row(m2['b2']))
        g = jnp.maximum(u_a[src] + v_a[dst], 0.0)
        em = c['edge_mlp']
        w1e = jnp.pad(em['W1'], ((0, 0), (0, 128 - em['W1'].shape[1])))
        msg = _edge_stage(g, ea_pad, w1e, row(em['b1']), em['W2'],
                          row(em['b2']), m1['W2'], row(m1['b2']))
        segmax = jax.ops.segment_max(msg, dst, num_segments=N)
        agg = jnp.where(has_in, segmax, 0.0)
        return b_a.at[:N].add(agg)

    z1 = conv_layer(z0, params['layer0'])

    a = params['att0']
    wkqvs = jnp.concatenate([a['Wk'], a['Wq'], a['Wv'], a['Wskip']], 0)
    bkqvs = jnp.concatenate([a['bk'], a['bq'], a['bv'], a['bias']])
    kqvs = _mm(z1, wkqvs, row(bkqvs))
    k = kqvs[:N, :D]
    q = kqvs[:N, D:2 * D]
    v = kqvs[:N, 2 * D:3 * D]
    m = jax.nn.sigmoid(k[dst] + q[src]) * v[src]
    s = jax.ops.segment_sum(m, dst, num_segments=N)
    gated = s / jnp.maximum(cnt, 1.0)[:, None]
    z2 = kqvs.at[:N, 3 * D:].add(gated)[:, 3 * D:]

    z3 = conv_layer(z2, params['layer1'])

    wc = jnp.concatenate([params['lin_u_W'][:, :D], params['lin_h_W'][:, :D]], 0)
    bc = jnp.concatenate([params['lin_u_b'], params['lin_h_b']])
    out = _final_stage(z3, params['lin2_W'], row(params['lin2_b']), wc, row(bc))
    return out[:N]


# speedup vs baseline: 1.3077x; 1.3077x over previous
import functools
import jax
import jax.numpy as jnp
from jax import lax
from jax.experimental import pallas as pl
from jax.experimental.pallas import tpu as pltpu
from jax.experimental.pallas import tpu_sc as plsc

N = 10000
E = 160000
D = 256
NPAD = 10240
ROW_BLK = 256
EDGE_BLK = 1280
F32 = jnp.float32

NW = 32
ECHUNK = 128
NCHUNKS = E // ECHUNK
CHUNKS_PER_TILE = NCHUNKS // NW
CHUNKS_REM = NCHUNKS - CHUNKS_PER_TILE * NW

_SC_MESH = plsc.VectorSubcoreMesh(core_axis_name="c", subcore_axis_name="s")


def _wid():
    return lax.axis_index("s") * 2 + lax.axis_index("c")


def _gather_body(u_hbm, v_hbm, src_hbm, dst_hbm, out_hbm,
                 sidx, didx, bufu, bufv, semu, semv):
    wid = _wid()

    def do_chunk(cid):
        e0 = cid * ECHUNK
        pltpu.sync_copy(src_hbm.at[pl.ds(e0, ECHUNK)], sidx)
        pltpu.sync_copy(dst_hbm.at[pl.ds(e0, ECHUNK)], didx)
        cu = pltpu.async_copy(u_hbm.at[sidx], bufu, semu)
        cv = pltpu.async_copy(v_hbm.at[didx], bufv, semv)
        cu.wait()
        cv.wait()

        def row(r, _):
            for c in range(D // 16):
                s = pl.ds(c * 16, 16)
                bufu[r, s] = jnp.maximum(bufu[r, s] + bufv[r, s], 0.0)
            return 0

        lax.fori_loop(0, ECHUNK, row, 0, unroll=False)
        pltpu.sync_copy(bufu, out_hbm.at[pl.ds(e0, ECHUNK), :])

    def chunk_loop(j, _):
        do_chunk(j * NW + wid)
        return 0

    lax.fori_loop(0, CHUNKS_PER_TILE, chunk_loop, 0, unroll=False)

    @pl.when(wid < CHUNKS_REM)
    def _():
        do_chunk(CHUNKS_PER_TILE * NW + wid)


def _sc_gather_relu_add(u, v, src, dst):
    return pl.kernel(
        _gather_body,
        out_type=jax.ShapeDtypeStruct((E, D), F32),
        mesh=_SC_MESH,
        scratch_types=[
            pltpu.VMEM((ECHUNK,), jnp.int32),
            pltpu.VMEM((ECHUNK,), jnp.int32),
            pltpu.VMEM((ECHUNK, D), F32),
            pltpu.VMEM((ECHUNK, D), F32),
            pltpu.SemaphoreType.DMA,
            pltpu.SemaphoreType.DMA,
        ],
    )(u, v, src, dst)


def _dot_t(a, w):
    return lax.dot_general(a, w, (((1,), (1,)), ((), ())),
                           preferred_element_type=F32)


def _mm_body(x_ref, w_ref, b_ref, o_ref):
    o_ref[...] = _dot_t(x_ref[...], w_ref[...]) + b_ref[...]


def _mm(x, w, b):
    rows, k = x.shape
    out_d = w.shape[0]
    return pl.pallas_call(
        _mm_body,
        grid=(rows // ROW_BLK,),
        in_specs=[
            pl.BlockSpec((ROW_BLK, k), lambda i: (i, 0)),
            pl.BlockSpec((out_d, k), lambda i: (0, 0)),
            pl.BlockSpec((1, out_d), lambda i: (0, 0)),
        ],
        out_specs=pl.BlockSpec((ROW_BLK, out_d), lambda i: (i, 0)),
        out_shape=jax.ShapeDtypeStruct((rows, out_d), F32),
    )(x, w, b)


def _node_body(z_ref, g_ref, bln_ref, wc1_ref, bc1_ref, w2_ref, b2_ref,
               u_ref, v_ref, bout_ref):
    z = z_ref[...]
    mu = jnp.mean(z, axis=1, keepdims=True)
    zc = z - mu
    var = jnp.mean(zc * zc, axis=1, keepdims=True)
    h = jnp.maximum(zc * lax.rsqrt(var + 1e-5) * g_ref[...] + bln_ref[...], 0.0)
    cat = _dot_t(h, wc1_ref[...]) + bc1_ref[...]
    u_ref[...] = cat[:, :D]
    v_ref[...] = cat[:, D:2 * D]
    t = jnp.maximum(cat[:, 2 * D:], 0.0)
    bout_ref[...] = z + _dot_t(t, w2_ref[...]) + b2_ref[...]


def _node_stage(z, ln_g, ln_b, wc1, bc1, w2, b2):
    rows = z.shape[0]
    return pl.pallas_call(
        _node_body,
        grid=(rows // ROW_BLK,),
        in_specs=[
            pl.BlockSpec((ROW_BLK, D), lambda i: (i, 0)),
            pl.BlockSpec((1, D), lambda i: (0, 0)),
            pl.BlockSpec((1, D), lambda i: (0, 0)),
            pl.BlockSpec((3 * D, D), lambda i: (0, 0)),
            pl.BlockSpec((1, 3 * D), lambda i: (0, 0)),
            pl.BlockSpec((D, D), lambda i: (0, 0)),
            pl.BlockSpec((1, D), lambda i: (0, 0)),
        ],
        out_specs=[
            pl.BlockSpec((ROW_BLK, D), lambda i: (i, 0)),
            pl.BlockSpec((ROW_BLK, D), lambda i: (i, 0)),
            pl.BlockSpec((ROW_BLK, D), lambda i: (i, 0)),
        ],
        out_shape=[
            jax.ShapeDtypeStruct((rows, D), F32),
            jax.ShapeDtypeStruct((rows, D), F32),
            jax.ShapeDtypeStruct((rows, D), F32),
        ],
    )(z, ln_g, ln_b, wc1, bc1, w2, b2)


def _edge_body(g_ref, ea_ref, w1e_ref, b1e_ref, w2e_ref, b2e_ref,
               w2m_ref, b2m_ref, o_ref):
    gh = jnp.maximum(_dot_t(ea_ref[...], w1e_ref[...]) + b1e_ref[...], 0.0)
    gate = _dot_t(gh, w2e_ref[...]) + b2e_ref[...]
    msgpre = _dot_t(g_ref[...], w2m_ref[...]) + b2m_ref[...]
    o_ref[...] = gate * msgpre


def _edge_stage(g, ea_pad, w1e, b1e, w2e, b2e, w2m, b2m):
    ek = ea_pad.shape[1]
    return pl.pallas_call(
        _edge_body,
        grid=(E // EDGE_BLK,),
        in_specs=[
            pl.BlockSpec((EDGE_BLK, D), lambda i: (i, 0)),
            pl.BlockSpec((EDGE_BLK, ek), lambda i: (i, 0)),
            pl.BlockSpec((D, ek), lambda i: (0, 0)),
            pl.BlockSpec((1, D), lambda i: (0, 0)),
            pl.BlockSpec((D, D), lambda i: (0, 0)),
            pl.BlockSpec((1, D), lambda i: (0, 0)),
            pl.BlockSpec((D, D), lambda i: (0, 0)),
            pl.BlockSpec((1, D), lambda i: (0, 0)),
        ],
        out_specs=pl.BlockSpec((EDGE_BLK, D), lambda i: (i, 0)),
        out_shape=jax.ShapeDtypeStruct((E, D), F32),
    )(g, ea_pad, w1e, b1e, w2e, b2e, w2m, b2m)


def _final_body(z_ref, w_ref, b_ref, wc_ref, bc_ref, o_ref):
    gc = _dot_t(z_ref[...], w_ref[...]) + b_ref[...]
    cat = _dot_t(gc, wc_ref[...]) + bc_ref[...]
    u = jax.nn.sigmoid(cat[:, :D])
    hc = jnp.maximum(cat[:, D:], 0.0)
    o_ref[...] = (1.0 - u) * hc


def _final_stage(z, w, b, wc, bc):
    rows = z.shape[0]
    return pl.pallas_call(
        _final_body,
        grid=(rows // ROW_BLK,),
        in_specs=[
            pl.BlockSpec((ROW_BLK, D), lambda i: (i, 0)),
            pl.BlockSpec((D, D), lambda i: (0, 0)),
            pl.BlockSpec((1, D), lambda i: (0, 0)),
            pl.BlockSpec((2 * D, D), lambda i: (0, 0)),
            pl.BlockSpec((1, 2 * D), lambda i: (0, 0)),
        ],
        out_specs=pl.BlockSpec((ROW_BLK, D), lambda i: (i, 0)),
        out_shape=jax.ShapeDtypeStruct((rows, D), F32),
    )(z, w, b, wc, bc)


def kernel(x, edge_index, edge_attr, params):
    src = edge_index[0]
    dst = edge_index[1]

    x_pad = jnp.pad(x, ((0, NPAD - N), (0, 0)))
    ea_pad = jnp.pad(edge_attr, ((0, 0), (0, 128 - edge_attr.shape[1])))

    cnt = jax.ops.segment_sum(jnp.ones((E,), F32), dst, num_segments=N)
    has_in = (cnt > 0)[:, None]

    def row(v):
        return v.reshape(1, -1)

    z0 = _mm(x_pad, params['lin1_W'], row(params['lin1_b']))

    def conv_layer(z, p):
        c = p['conv']
        m1 = c['node_mlp1']
        m2 = c['node_mlp2']
        wc1 = jnp.concatenate([m1['W1'][:, :D], m1['W1'][:, D:], m2['W1']], 0)
        bc1 = jnp.concatenate([m1['b1'], jnp.zeros((D,), F32), m2['b1']])
        u_a, v_a, b_a = _node_stage(z, row(p['ln_g']), row(p['ln_b']),
                                    wc1, row(bc1), m2['W2'], row(m2['b2']))
        g = _sc_gather_relu_add(u_a, v_a, src, dst)
        em = c['edge_mlp']
        w1e = jnp.pad(em['W1'], ((0, 0), (0, 128 - em['W1'].shape[1])))
        msg = _edge_stage(g, ea_pad, w1e, row(em['b1']), em['W2'],
                          row(em['b2']), m1['W2'], row(m1['b2']))
        segmax = jax.ops.segment_max(msg, dst, num_segments=N)
        agg = jnp.where(has_in, segmax, 0.0)
        return b_a.at[:N].add(agg)

    z1 = conv_layer(z0, params['layer0'])

    a = params['att0']
    wkqvs = jnp.concatenate([a['Wk'], a['Wq'], a['Wv'], a['Wskip']], 0)
    bkqvs = jnp.concatenate([a['bk'], a['bq'], a['bv'], a['bias']])
    kqvs = _mm(z1, wkqvs, row(bkqvs))
    k = kqvs[:N, :D]
    q = kqvs[:N, D:2 * D]
    v = kqvs[:N, 2 * D:3 * D]
    m = jax.nn.sigmoid(k[dst] + q[src]) * v[src]
    s = jax.ops.segment_sum(m, dst, num_segments=N)
    gated = s / jnp.maximum(cnt, 1.0)[:, None]
    z2 = kqvs.at[:N, 3 * D:].add(gated)[:, 3 * D:]

    z3 = conv_layer(z2, params['layer1'])

    wc = jnp.concatenate([params['lin_u_W'][:, :D], params['lin_h_W'][:, :D]], 0)
    bc = jnp.concatenate([params['lin_u_b'], params['lin_h_b']])
    out = _final_stage(z3, params['lin2_W'], row(params['lin2_b']), wc, row(bc))
    return out[:N]


# speedup vs baseline: 1.3079x; 1.0001x over previous
import functools
import jax
import jax.numpy as jnp
from jax import lax
from jax.experimental import pallas as pl
from jax.experimental.pallas import tpu as pltpu
from jax.experimental.pallas import tpu_sc as plsc

N = 10000
E = 160000
D = 256
NPAD = 10240
ROW_BLK = 256
EDGE_BLK = 1280
F32 = jnp.float32

NW = 32
ECHUNK = 128
NCHUNKS = E // ECHUNK
CHUNKS_PER_TILE = NCHUNKS // NW
CHUNKS_REM = NCHUNKS - CHUNKS_PER_TILE * NW

_SC_MESH = plsc.VectorSubcoreMesh(core_axis_name="c", subcore_axis_name="s")


def _wid():
    return lax.axis_index("s") * 2 + lax.axis_index("c")


def _gather_body(u_hbm, v_hbm, src_hbm, dst_hbm, out_hbm,
                 sidx, didx, bufu, bufv, semu, semv):
    wid = _wid()

    def do_chunk(cid):
        e0 = cid * ECHUNK
        pltpu.sync_copy(src_hbm.at[pl.ds(e0, ECHUNK)], sidx)
        pltpu.sync_copy(dst_hbm.at[pl.ds(e0, ECHUNK)], didx)
        cu = pltpu.async_copy(u_hbm.at[sidx], bufu, semu)
        cv = pltpu.async_copy(v_hbm.at[didx], bufv, semv)
        cu.wait()
        cv.wait()

        def row(r, _):
            for c in range(D // 16):
                s = pl.ds(c * 16, 16)
                bufu[r, s] = jnp.maximum(bufu[r, s] + bufv[r, s], 0.0)
            return 0

        lax.fori_loop(0, ECHUNK, row, 0, unroll=False)
        pltpu.sync_copy(bufu, out_hbm.at[pl.ds(e0, ECHUNK), :])

    def chunk_loop(j, _):
        do_chunk(j * NW + wid)
        return 0

    lax.fori_loop(0, CHUNKS_PER_TILE, chunk_loop, 0, unroll=False)

    @pl.when(wid < CHUNKS_REM)
    def _():
        do_chunk(CHUNKS_PER_TILE * NW + wid)


def _sc_gather_relu_add(u, v, src, dst):
    return pl.kernel(
        _gather_body,
        out_type=jax.ShapeDtypeStruct((E, D), F32),
        mesh=_SC_MESH,
        scratch_types=[
            pltpu.VMEM((ECHUNK,), jnp.int32),
            pltpu.VMEM((ECHUNK,), jnp.int32),
            pltpu.VMEM((ECHUNK, D), F32),
            pltpu.VMEM((ECHUNK, D), F32),
            pltpu.SemaphoreType.DMA,
            pltpu.SemaphoreType.DMA,
        ],
    )(u, v, src, dst)


def _dot_t(a, w):
    return lax.dot_general(a, w, (((1,), (1,)), ((), ())),
                           preferred_element_type=F32)


def _medge_body(k_hbm, qv_hbm, src_hbm, dst_hbm, out_hbm,
                sidx, didx, bufk, bufqv, semk, semqv):
    wid = _wid()

    def do_chunk(cid):
        e0 = cid * ECHUNK
        pltpu.sync_copy(src_hbm.at[pl.ds(e0, ECHUNK)], sidx)
        pltpu.sync_copy(dst_hbm.at[pl.ds(e0, ECHUNK)], didx)
        ck = pltpu.async_copy(k_hbm.at[didx], bufk, semk)
        cqv = pltpu.async_copy(qv_hbm.at[sidx], bufqv, semqv)
        ck.wait()
        cqv.wait()

        def row(r, _):
            for c in range(D // 16):
                sl = pl.ds(c * 16, 16)
                t = bufk[r, sl] + bufqv[r, sl]
                sig = 1.0 / (1.0 + jnp.exp(-t))
                bufk[r, sl] = sig * bufqv[r, pl.ds(D + c * 16, 16)]
            return 0

        lax.fori_loop(0, ECHUNK, row, 0, unroll=False)
        pltpu.sync_copy(bufk, out_hbm.at[pl.ds(e0, ECHUNK), :])

    def chunk_loop(j, _):
        do_chunk(j * NW + wid)
        return 0

    lax.fori_loop(0, CHUNKS_PER_TILE, chunk_loop, 0, unroll=False)

    @pl.when(wid < CHUNKS_REM)
    def _():
        do_chunk(CHUNKS_PER_TILE * NW + wid)


def _sc_medge(k, qv, src, dst):
    return pl.kernel(
        _medge_body,
        out_type=jax.ShapeDtypeStruct((E, D), F32),
        mesh=_SC_MESH,
        scratch_types=[
            pltpu.VMEM((ECHUNK,), jnp.int32),
            pltpu.VMEM((ECHUNK,), jnp.int32),
            pltpu.VMEM((ECHUNK, D), F32),
            pltpu.VMEM((ECHUNK, 2 * D), F32),
            pltpu.SemaphoreType.DMA,
            pltpu.SemaphoreType.DMA,
        ],
    )(k, qv, src, dst)


IDXBLK = 2048
NBLK = E // IDXBLK
NPASS = 2
NODES_PER = NPAD // (NW * NPASS)
ACCROWS = NODES_PER + 16
LISTCAP = IDXBLK + 64
DRAIN = 32
NEG = -3.0e38


def _gather16(x, idx):
    dnums = lax.GatherDimensionNumbers(
        offset_dims=(), collapsed_slice_dims=(0,), start_index_map=(0,))
    return lax.gather(x, idx[:, None], dnums, (1,),
                      mode=lax.GatherScatterMode.PROMISE_IN_BOUNDS)


def _cumsum16(x, lane):
    cum = x
    for k in (1, 2, 4, 8):
        shifted = _gather16(cum, jnp.maximum(lane - k, 0))
        cum = cum + jnp.where(lane >= k, shifted, 0)
    return cum


def _compact_perm(cum, lane):
    lo = jnp.zeros((16,), jnp.int32)
    for k in (8, 4, 2, 1):
        mid = lo + k
        cv = _gather16(cum, mid - 1)
        lo = jnp.where(cv < lane + 1, mid, lo)
    return jnp.minimum(lo, 15)


def _make_seg_body(mode):
    def body(*refs):
        if mode == "sum":
            (dst_hbm, val_hbm, aux_hbm, inv_hbm, out_hbm,
             idxblk, elist, olist, bufrows, owin, abuf, ibuf, acc, sem) = refs
        else:
            (dst_hbm, val_hbm, aux_hbm, out_hbm,
             idxblk, elist, olist, bufrows, owin, osm, acc, sem) = refs
        wid = _wid()
        s = lax.axis_index("s")
        lane = lax.iota(jnp.int32, 16)
        init = 0.0 if mode == "sum" else NEG
        rbase = s * ACCROWS if mode == "sum" else 0

        for p in range(NPASS):
            base = (p * NW + wid) * NODES_PER

            if mode == "sum":
                def zrow(r, _):
                    for cc in range(D // 16):
                        bufrows[r, pl.ds(cc * 16, 16)] = jnp.zeros((16,), F32)
                    return 0

                lax.fori_loop(0, DRAIN, zrow, 0, unroll=False)
                for j in range(ACCROWS // DRAIN):
                    pltpu.sync_copy(bufrows,
                                    acc.at[pl.ds(rbase + j * DRAIN, DRAIN), :])
                pltpu.sync_copy(
                    bufrows.at[pl.ds(0, ACCROWS % DRAIN), :],
                    acc.at[pl.ds(rbase + (ACCROWS // DRAIN) * DRAIN,
                                 ACCROWS % DRAIN), :])
            else:
                def irow(r, _):
                    for cc in range(D // 16):
                        acc[r, pl.ds(cc * 16, 16)] = jnp.full((16,), init, F32)
                    return 0

                lax.fori_loop(0, ACCROWS, irow, 0, unroll=False)

            def block(blk, _):
                e0 = blk * IDXBLK
                pltpu.sync_copy(dst_hbm.at[pl.ds(e0, IDXBLK)], idxblk)

                def compact(v, ptr):
                    d = idxblk[pl.ds(v * 16, 16)]
                    off = d - base
                    mask = (off >= 0) & (off < NODES_PER)
                    cum = _cumsum16(jnp.where(mask, 1, 0), lane)
                    eid = e0 + v * 16 + lane
                    perm = _compact_perm(cum, lane)
                    eid_s = _gather16(eid, perm)
                    off_s = _gather16(off, perm) + rbase
                    elist[pl.ds(ptr, 16)] = eid_s
                    olist[pl.ds(ptr, 16)] = off_s
                    return ptr + cum[15]

                ptr = lax.fori_loop(0, IDXBLK // 16, compact, 0, unroll=False)

                p0 = ptr & ~15

                for t in range(2):
                    bt = p0 + t * 16
                    keep = (bt + lane) < ptr
                    ev = elist[pl.ds(bt, 16)]
                    ov = olist[pl.ds(bt, 16)]
                    elist[pl.ds(bt, 16)] = jnp.where(keep, ev, 0)
                    olist[pl.ds(bt, 16)] = jnp.where(keep, ov,
                                                     rbase + NODES_PER)

                ndrain = (ptr + DRAIN - 1) >> 5

                def drain(dd, _):
                    ev0 = elist[pl.ds(dd * DRAIN, 16)]
                    ev1 = elist[pl.ds(dd * DRAIN + 16, 16)]
                    cp0 = pltpu.async_copy(
                        val_hbm.at[ev0], bufrows.at[pl.ds(0, 16), :], sem)
                    cp1 = pltpu.async_copy(
                        val_hbm.at[ev1], bufrows.at[pl.ds(16, 16), :], sem)
                    cp0.wait()
                    cp1.wait()
                    if mode == "sum":
                        ov0 = olist[pl.ds(dd * DRAIN, 16)]
                        ov1 = olist[pl.ds(dd * DRAIN + 16, 16)]
                        pltpu.sync_copy(bufrows.at[pl.ds(0, 16), :],
                                        acc.at[ov0], add=True)
                        pltpu.sync_copy(bufrows.at[pl.ds(16, 16), :],
                                        acc.at[ov1], add=True)
                    else:
                        for h in range(DRAIN // 16):
                            ov = olist[pl.ds(dd * DRAIN + h * 16, 16)]
                            for i in range(16):
                                osm[h * 16 + i] = ov[i]

                        def rmw(ii, _):
                            o = osm[ii]
                            for cc in range(D // 16):
                                sl = pl.ds(cc * 16, 16)
                                acc[o, sl] = jnp.maximum(acc[o, sl],
                                                         bufrows[ii, sl])
                            return 0

                        lax.fori_loop(0, DRAIN, rmw, 0, unroll=False)
                    return 0

                lax.fori_loop(0, ndrain, drain, 0, unroll=False)
                return 0

            lax.fori_loop(0, NBLK, block, 0, unroll=False)

            for ch in range(NODES_PER // DRAIN):
                r0 = ch * DRAIN
                pltpu.sync_copy(aux_hbm.at[pl.ds(base + r0, DRAIN), :],
                                bufrows)
                if mode == "sum":
                    pltpu.sync_copy(acc.at[pl.ds(rbase + r0, DRAIN), :], abuf)
                    pltpu.sync_copy(inv_hbm.at[pl.ds(base + r0, DRAIN), :],
                                    ibuf)

                def erow(r, _):
                    for cc in range(D // 16):
                        sl = pl.ds(cc * 16, 16)
                        xv = bufrows[r, sl]
                        if mode == "sum":
                            iv = ibuf[r, pl.ds((cc % 8) * 16, 16)]
                            bufrows[r, sl] = abuf[r, sl] * iv + xv
                        else:
                            av = acc[r0 + r, sl]
                            bufrows[r, sl] = xv + jnp.where(av > NEG / 2,
                                                            av, 0.0)
                    return 0

                lax.fori_loop(0, DRAIN, erow, 0, unroll=False)
                pltpu.sync_copy(bufrows, out_hbm.at[pl.ds(base + r0, DRAIN), :])

    return body


_seg_body_max = _make_seg_body("max")
_seg_body_sum = _make_seg_body("sum")


def _sc_segment_combine(mode, dst, val, aux, inv=None):
    common = [
        pltpu.VMEM((IDXBLK,), jnp.int32),
        pltpu.VMEM((LISTCAP,), jnp.int32),
        pltpu.VMEM((LISTCAP,), jnp.int32),
        pltpu.VMEM((DRAIN, D), F32),
        pltpu.VMEM((DRAIN,), jnp.int32),
    ]
    if mode == "sum":
        scratch = common + [
            pltpu.VMEM((DRAIN, D), F32),
            pltpu.VMEM((DRAIN, 128), F32),
            pltpu.VMEM_SHARED((16 * ACCROWS, D), F32),
            pltpu.SemaphoreType.DMA,
        ]
        args = (dst, val, aux, inv)
    else:
        scratch = common + [
            pltpu.SMEM((DRAIN,), jnp.int32),
            pltpu.VMEM((ACCROWS, D), F32),
            pltpu.SemaphoreType.DMA,
        ]
        args = (dst, val, aux)
    return pl.kernel(
        _seg_body_sum if mode == "sum" else _seg_body_max,
        out_type=jax.ShapeDtypeStruct((NPAD, D), F32),
        mesh=_SC_MESH,
        scratch_types=scratch,
    )(*args)


def _mm_body(x_ref, w_ref, b_ref, o_ref):
    o_ref[...] = _dot_t(x_ref[...], w_ref[...]) + b_ref[...]


def _mm(x, w, b):
    rows, k = x.shape
    out_d = w.shape[0]
    return pl.pallas_call(
        _mm_body,
        grid=(rows // ROW_BLK,),
        in_specs=[
            pl.BlockSpec((ROW_BLK, k), lambda i: (i, 0)),
            pl.BlockSpec((out_d, k), lambda i: (0, 0)),
            pl.BlockSpec((1, out_d), lambda i: (0, 0)),
        ],
        out_specs=pl.BlockSpec((ROW_BLK, out_d), lambda i: (i, 0)),
        out_shape=jax.ShapeDtypeStruct((rows, out_d), F32),
    )(x, w, b)


def _node_body(z_ref, g_ref, bln_ref, wc1_ref, bc1_ref, w2_ref, b2_ref,
               u_ref, v_ref, bout_ref):
    z = z_ref[...]
    mu = jnp.mean(z, axis=1, keepdims=True)
    zc = z - mu
    var = jnp.mean(zc * zc, axis=1, keepdims=True)
    h = jnp.maximum(zc * lax.rsqrt(var + 1e-5) * g_ref[...] + bln_ref[...], 0.0)
    cat = _dot_t(h, wc1_ref[...]) + bc1_ref[...]
    u_ref[...] = cat[:, :D]
    v_ref[...] = cat[:, D:2 * D]
    t = jnp.maximum(cat[:, 2 * D:], 0.0)
    bout_ref[...] = z + _dot_t(t, w2_ref[...]) + b2_ref[...]


def _node_stage(z, ln_g, ln_b, wc1, bc1, w2, b2):
    rows = z.shape[0]
    return pl.pallas_call(
        _node_body,
        grid=(rows // ROW_BLK,),
        in_specs=[
            pl.BlockSpec((ROW_BLK, D), lambda i: (i, 0)),
            pl.BlockSpec((1, D), lambda i: (0, 0)),
            pl.BlockSpec((1, D), lambda i: (0, 0)),
            pl.BlockSpec((3 * D, D), lambda i: (0, 0)),
            pl.BlockSpec((1, 3 * D), lambda i: (0, 0)),
            pl.BlockSpec((D, D), lambda i: (0, 0)),
            pl.BlockSpec((1, D), lambda i: (0, 0)),
        ],
        out_specs=[
            pl.BlockSpec((ROW_BLK, D), lambda i: (i, 0)),
            pl.BlockSpec((ROW_BLK, D), lambda i: (i, 0)),
            pl.BlockSpec((ROW_BLK, D), lambda i: (i, 0)),
        ],
        out_shape=[
            jax.ShapeDtypeStruct((rows, D), F32),
            jax.ShapeDtypeStruct((rows, D), F32),
            jax.ShapeDtypeStruct((rows, D), F32),
        ],
    )(z, ln_g, ln_b, wc1, bc1, w2, b2)


def _edge_body(g_ref, ea_ref, w1e_ref, b1e_ref, w2e_ref, b2e_ref,
               w2m_ref, b2m_ref, o_ref):
    gh = jnp.maximum(_dot_t(ea_ref[...], w1e_ref[...]) + b1e_ref[...], 0.0)
    gate = _dot_t(gh, w2e_ref[...]) + b2e_ref[...]
    msgpre = _dot_t(g_ref[...], w2m_ref[...]) + b2m_ref[...]
    o_ref[...] = gate * msgpre


def _edge_stage(g, ea_pad, w1e, b1e, w2e, b2e, w2m, b2m):
    ek = ea_pad.shape[1]
    return pl.pallas_call(
        _edge_body,
        grid=(E // EDGE_BLK,),
        in_specs=[
            pl.BlockSpec((EDGE_BLK, D), lambda i: (i, 0)),
            pl.BlockSpec((EDGE_BLK, ek), lambda i: (i, 0)),
            pl.BlockSpec((D, ek), lambda i: (0, 0)),
            pl.BlockSpec((1, D), lambda i: (0, 0)),
            pl.BlockSpec((D, D), lambda i: (0, 0)),
            pl.BlockSpec((1, D), lambda i: (0, 0)),
            pl.BlockSpec((D, D), lambda i: (0, 0)),
            pl.BlockSpec((1, D), lambda i: (0, 0)),
        ],
        out_specs=pl.BlockSpec((EDGE_BLK, D), lambda i: (i, 0)),
        out_shape=jax.ShapeDtypeStruct((E, D), F32),
    )(g, ea_pad, w1e, b1e, w2e, b2e, w2m, b2m)


def _final_body(z_ref, w_ref, b_ref, wc_ref, bc_ref, o_ref):
    gc = _dot_t(z_ref[...], w_ref[...]) + b_ref[...]
    cat = _dot_t(gc, wc_ref[...]) + bc_ref[...]
    u = jax.nn.sigmoid(cat[:, :D])
    hc = jnp.maximum(cat[:, D:], 0.0)
    o_ref[...] = (1.0 - u) * hc


def _final_stage(z, w, b, wc, bc):
    rows = z.shape[0]
    return pl.pallas_call(
        _final_body,
        grid=(rows // ROW_BLK,),
        in_specs=[
            pl.BlockSpec((ROW_BLK, D), lambda i: (i, 0)),
            pl.BlockSpec((D, D), lambda i: (0, 0)),
            pl.BlockSpec((1, D), lambda i: (0, 0)),
            pl.BlockSpec((2 * D, D), lambda i: (0, 0)),
            pl.BlockSpec((1, 2 * D), lambda i: (0, 0)),
        ],
        out_specs=pl.BlockSpec((ROW_BLK, D), lambda i: (i, 0)),
        out_shape=jax.ShapeDtypeStruct((rows, D), F32),
    )(z, w, b, wc, bc)


def kernel(x, edge_index, edge_attr, params):
    src = edge_index[0]
    dst = edge_index[1]

    x_pad = jnp.pad(x, ((0, NPAD - N), (0, 0)))
    ea_pad = jnp.pad(edge_attr, ((0, 0), (0, 128 - edge_attr.shape[1])))

    cnt = jax.ops.segment_sum(jnp.ones((E,), F32), dst, num_segments=N)
    has_in = (cnt > 0)[:, None]

    def row(v):
        return v.reshape(1, -1)

    z0 = _mm(x_pad, params['lin1_W'], row(params['lin1_b']))

    def conv_layer(z, p):
        c = p['conv']
        m1 = c['node_mlp1']
        m2 = c['node_mlp2']
        wc1 = jnp.concatenate([m1['W1'][:, :D], m1['W1'][:, D:], m2['W1']], 0)
        bc1 = jnp.concatenate([m1['b1'], jnp.zeros((D,), F32), m2['b1']])
        u_a, v_a, b_a = _node_stage(z, row(p['ln_g']), row(p['ln_b']),
                                    wc1, row(bc1), m2['W2'], row(m2['b2']))
        g = _sc_gather_relu_add(u_a, v_a, src, dst)
        em = c['edge_mlp']
        w1e = jnp.pad(em['W1'], ((0, 0), (0, 128 - em['W1'].shape[1])))
        msg = _edge_stage(g, ea_pad, w1e, row(em['b1']), em['W2'],
                          row(em['b2']), m1['W2'], row(m1['b2']))
        segmax = jax.ops.segment_max(msg, dst, num_segments=N)
        agg = jnp.where(has_in, segmax, 0.0)
        return b_a.at[:N].add(agg)

    z1 = conv_layer(z0, params['layer0'])

    a = params['att0']
    wkqvs = jnp.concatenate([a['Wk'], a['Wq'], a['Wv'], a['Wskip']], 0)
    bkqvs = jnp.concatenate([a['bk'], a['bq'], a['bv'], a['bias']])
    kqvs = _mm(z1, wkqvs, row(bkqvs))
    k = kqvs[:N, :D]
    q = kqvs[:N, D:2 * D]
    v = kqvs[:N, 2 * D:3 * D]
    m = jax.nn.sigmoid(k[dst] + q[src]) * v[src]
    s = jax.ops.segment_sum(m, dst, num_segments=N)
    gated = s / jnp.maximum(cnt, 1.0)[:, None]
    z2 = kqvs.at[:N, 3 * D:].add(gated)[:, 3 * D:]

    z3 = conv_layer(z2, params['layer1'])

    wc = jnp.concatenate([params['lin_u_W'][:, :D], params['lin_h_W'][:, :D]], 0)
    bc = jnp.concatenate([params['lin_u_b'], params['lin_h_b']])
    out = _final_stage(z3, params['lin2_W'], row(params['lin2_b']), wc, row(bc))
    return out[:N]
